# baseline (speedup 1.0000x reference)
import functools

import jax
import jax.numpy as jnp
from jax import lax
from jax.experimental import pallas as pl
from jax.experimental.pallas import tpu as pltpu
from jax.experimental.pallas import tpu_sc as plsc

N = 8388608
NC = 2
NS = 16
NW = NC * NS
C = N // NW
WIN = 4096
NWIN = C // WIN
VPW = WIN // 16
R = 256
OBR = 512
SIDE = 32
NSIDE = R * SIDE
CCAP = 4096
NROW = N // 16
DUMPROW = NROW
DUMP = N

_SC_PARAMS = pltpu.CompilerParams(needs_layout_passes=False,
                                  use_tc_tiling_on_sc=False)


def _mesh():
    return plsc.VectorSubcoreMesh(core_axis_name="c", subcore_axis_name="s")


def _pack_body(k_ref, o_ref):
    b = lax.bitcast_convert_type(k_ref[...], jnp.uint32)
    flipped = jnp.where(
        (b >> 31) != 0, ~b, b | jnp.uint32(0x80000000))
    packed = jnp.where(
        (b & jnp.uint32(0x7FFFFFFF)) == 0, jnp.uint32(0x80000000), flipped)
    o_ref[...] = lax.bitcast_convert_type(packed, jnp.int32)


def _unpack_body(b_ref, o_ref):
    b = lax.bitcast_convert_type(b_ref[...], jnp.uint32)
    bits = jnp.where((b & jnp.uint32(0x80000000)) != 0,
                     b ^ jnp.uint32(0x80000000), ~b)
    o_ref[...] = lax.bitcast_convert_type(bits, jnp.float32)


def _elementwise_tc(body, x, out_dtype):
    x2 = x.reshape(8192, 1024)
    out = pl.pallas_call(
        body,
        out_shape=jax.ShapeDtypeStruct((8192, 1024), out_dtype),
        grid=(8,),
        in_specs=[pl.BlockSpec((1024, 1024), lambda i: (i, 0))],
        out_specs=pl.BlockSpec((1024, 1024), lambda i: (i, 0)),
    )(x2)
    return out.reshape(N)


def _wid():
    return lax.axis_index("s") * NC + lax.axis_index("c")


def _hist0_kernel():
    @functools.partial(
        pl.kernel,
        out_type=(jax.ShapeDtypeStruct((NW * R,), jnp.int32),
                  jax.ShapeDtypeStruct((NW * NWIN * R,), jnp.int32)),
        mesh=_mesh(),
        scratch_types=[
            pltpu.VMEM((WIN,), jnp.int32),
            pltpu.VMEM((R,), jnp.int32),
            pltpu.VMEM((R,), jnp.int32),
        ],
        compiler_params=_SC_PARAMS,
    )
    def hist_k(kb_hbm, hist_hbm, whist_hbm, win, histv, whv):
        wid = _wid()
        base = wid * C
        zero16 = jnp.zeros((16,), jnp.int32)
        for i in range(R // 16):
            histv[pl.ds(i * 16, 16)] = zero16

        def win_body(w, _):
            pltpu.sync_copy(kb_hbm.at[pl.ds(base + w * WIN, WIN)], win)
            for i in range(R // 16):
                whv[pl.ds(i * 16, 16)] = zero16

            def vreg_body(i, _):
                x = win[pl.ds(i * 16, 16)]
                d = x & (R - 1)
                cnt, last = plsc.scan_count(d)
                plsc.addupdate_scatter(whv, [d], cnt, mask=last)
                return 0

            lax.fori_loop(0, VPW, vreg_body, 0)
            for i in range(R // 16):
                sl = pl.ds(i * 16, 16)
                histv[sl] = histv[sl] + whv[sl]
            pltpu.sync_copy(whv, whist_hbm.at[pl.ds((wid * NWIN + w) * R, R)])
            return 0

        lax.fori_loop(0, NWIN, win_body, 0)
        pltpu.sync_copy(histv, hist_hbm.at[pl.ds(wid * R, R)])

    return hist_k


def _cleanup_kernel(shift, want_hist):
    @functools.partial(
        pl.kernel,
        out_type=(jax.ShapeDtypeStruct((N + 16,), jnp.int32),
                  jax.ShapeDtypeStruct((N + 16,), jnp.float32),
                  jax.ShapeDtypeStruct((NW * R,), jnp.int32),
                  jax.ShapeDtypeStruct((NW * NWIN * R,), jnp.int32)),
        mesh=_mesh(),
        scratch_types=[
            pltpu.VMEM((WIN,), jnp.int32),
            pltpu.VMEM((WIN,), jnp.float32),
            pltpu.VMEM((NWIN + 1, R), jnp.int32),
            pltpu.VMEM((CCAP,), jnp.int32),
            pltpu.VMEM((CCAP,), jnp.int32),
            pltpu.VMEM((CCAP,), jnp.float32),
            pltpu.VMEM((CCAP,), jnp.int32),
            pltpu.VMEM((CCAP,), jnp.int32),
            pltpu.VMEM((CCAP,), jnp.float32),
            pltpu.VMEM((16,), jnp.int32),
        ],
        compiler_params=_SC_PARAMS,
    )
    def clean_k(kb_hbm, pv_hbm, sp_hbm, sk_hbm, sv_hbm,
                kb_out, pv_out, hist_hbm, whist_hbm,
                wk, wv, whv, cp, ck, cv, qp, qk, qv, qnv):
        wid = _wid()
        base = wid * C
        zero16 = jnp.zeros((16,), jnp.int32)
        lane = lax.iota(jnp.int32, 16)

        def win_body(w, _):
            pltpu.sync_copy(kb_hbm.at[pl.ds(base + w * WIN, WIN)], wk)
            pltpu.sync_copy(pv_hbm.at[pl.ds(base + w * WIN, WIN)], wv)
            for i in range(R // 16):
                whv[w, pl.ds(i * 16, 16)] = zero16

            def vreg_body(i, _):
                x = wk[pl.ds(i * 16, 16)]
                d = lax.shift_right_logical(x, shift) & (R - 1)
                cnt, last = plsc.scan_count(d)
                plsc.addupdate_scatter(whv.at[w], [d], cnt, mask=last)
                return 0

            if want_hist:
                lax.fori_loop(0, VPW, vreg_body, 0)
            pltpu.sync_copy(wk, kb_out.at[pl.ds(base + w * WIN, WIN)])
            pltpu.sync_copy(wv, pv_out.at[pl.ds(base + w * WIN, WIN)])
            return 0

        lax.fori_loop(0, NWIN, win_body, 0)

        qnv[...] = zero16
        for i in range(CCAP // 16):
            qp[pl.ds(i * 16, 16)] = jnp.full((16,), DUMP, jnp.int32)

        def flush():
            pltpu.sync_copy(qk, kb_out.at[qp])
            pltpu.sync_copy(qv, pv_out.at[qp])
            qnv[...] = zero16

        def chunk_body(s, _):
            pltpu.sync_copy(sp_hbm.at[pl.ds(s * CCAP, CCAP)], cp)
            pltpu.sync_copy(sk_hbm.at[pl.ds(s * CCAP, CCAP)], ck)
            pltpu.sync_copy(sv_hbm.at[pl.ds(s * CCAP, CCAP)], cv)

            def vreg_body(i, _):
                p = cp[pl.ds(i * 16, 16)]
                inr = jnp.logical_and(p >= base, p < base + C)
                pc = jnp.cumsum(jnp.where(inr, 1, 0)) - jnp.where(inr, 1, 0)
                qn = qnv[...]
                addr = qn + pc
                x = ck[pl.ds(i * 16, 16)]
                plsc.store_scatter(qp, [addr], p, mask=inr)
                plsc.store_scatter(qk, [addr], x, mask=inr)
                plsc.store_scatter(qv, [addr], cv[pl.ds(i * 16, 16)], mask=inr)
                nadd = jnp.sum(jnp.where(inr, 1, 0))
                qnv[...] = qn + nadd
                if want_hist:
                    widx = jnp.where(inr, (p - base) >> 12, NWIN)
                    d = lax.shift_right_logical(x, shift) & (R - 1)
                    cnt0, last0 = plsc.scan_count(widx)
                    plsc.addupdate_scatter(whv, [widx, jnp.zeros((16,), jnp.int32)],
                                           -cnt0, mask=jnp.logical_and(last0, inr))
                    key2 = widx * R + d
                    cnt1, last1 = plsc.scan_count(key2)
                    plsc.addupdate_scatter(whv, [widx, d], cnt1,
                                           mask=jnp.logical_and(last1, inr))

                @pl.when(jnp.max(qnv[...]) >= CCAP - 16)
                def _():
                    flush()
                return 0

            lax.fori_loop(0, (CCAP // 16), vreg_body, 0)
            return 0

        lax.fori_loop(0, (NW * NSIDE) // CCAP, chunk_body, 0)
        flush()

        if want_hist:
            def wh_out(w, _):
                pltpu.sync_copy(
                    whv.at[w], whist_hbm.at[pl.ds((wid * NWIN + w) * R, R)])
                return 0

            lax.fori_loop(0, NWIN, wh_out, 0)
            for i in range(R // 16):
                sl = pl.ds(i * 16, 16)
                acc = zero16

                def racc(w, a):
                    return a + whv[w, sl]

                acc = lax.fori_loop(0, NWIN, racc, acc)
                whv[NWIN, sl] = acc
            pltpu.sync_copy(whv.at[NWIN], hist_hbm.at[pl.ds(wid * R, R)])
        else:
            whv[NWIN, pl.ds(0, 16)] = zero16
            pltpu.sync_copy(whv.at[NWIN], hist_hbm.at[pl.ds(wid * R, R)])
            pltpu.sync_copy(
                whv.at[NWIN], whist_hbm.at[pl.ds(wid * NWIN * R, R)])

    return clean_k


def _permute_kernel(shift):
    @functools.partial(
        pl.kernel,
        out_type=(jax.ShapeDtypeStruct((NROW + 1, 16), jnp.int32),
                  jax.ShapeDtypeStruct((NROW + 1, 16), jnp.float32),
                  jax.ShapeDtypeStruct((NW, R, SIDE), jnp.int32),
                  jax.ShapeDtypeStruct((NW, R, SIDE), jnp.int32),
                  jax.ShapeDtypeStruct((NW, R, SIDE), jnp.float32)),
        mesh=_mesh(),
        scratch_types=[
            pltpu.VMEM((NW * R,), jnp.int32),
            pltpu.VMEM((R,), jnp.int32),
            pltpu.VMEM((R,), jnp.int32),
            pltpu.VMEM((R,), jnp.int32),
            pltpu.VMEM((R,), jnp.int32),
            pltpu.VMEM((R,), jnp.int32),
            pltpu.VMEM((R,), jnp.int32),
            pltpu.VMEM((R,), jnp.int32),
            pltpu.VMEM((WIN,), jnp.int32),
            pltpu.VMEM((WIN,), jnp.float32),
            pltpu.VMEM((OBR, 16), jnp.int32),
            pltpu.VMEM((OBR, 16), jnp.float32),
            pltpu.VMEM((OBR,), jnp.int32),
            pltpu.VMEM((R, 16), jnp.int32),
            pltpu.VMEM((R, 16), jnp.float32),
            pltpu.VMEM((R, SIDE), jnp.int32),
            pltpu.VMEM((R, SIDE), jnp.int32),
            pltpu.VMEM((R, SIDE), jnp.float32),
        ],
        compiler_params=_SC_PARAMS,
    )
    def perm_k(kb_hbm, pv_hbm, hist_hbm, whist_hbm,
               kb_out, pv_out, sp_out, sk_out, sv_out,
               histv, loffr, loff0r, aer, ber, vvr, omur, hcr,
               wk, wv, obk, obv, rowidx, crk, crv, sp, sk, sv):
        wid = _wid()
        base = wid * C
        zero16 = jnp.zeros((16,), jnp.int32)
        lane = lax.iota(jnp.int32, 16)
        pltpu.sync_copy(hist_hbm, histv)

        def db_body(db, carry):
            def w_body(w, ap):
                acc, pre = ap
                row = histv[pl.ds(w * R + db * 16, 16)]
                take = lax.broadcast(w < wid, (16,))
                return acc + row, pre + jnp.where(take, row, zero16)

            acc, pre = lax.fori_loop(0, NW, w_body, (zero16, zero16))
            l0 = carry + jnp.cumsum(acc) - acc + pre
            mine = histv[pl.ds(wid * R + db * 16, 16)]
            sl = pl.ds(db * 16, 16)
            loffr[sl] = l0
            loff0r[sl] = l0
            ae = (l0 + 15) & ~15
            end = l0 + mine
            aer[sl] = ae
            ber[sl] = jnp.maximum(ae, end & ~15)
            return carry + jnp.sum(acc)

        lax.fori_loop(0, R // 16, db_body, 0)

        for db in range(R // 16):
            sl = pl.ds(db * 16, 16)
            l0 = loff0r[sl]
            cme = histv[pl.ds(wid * R + db * 16, 16)]
            end = l0 + cme
            rt0 = l0 >> 4
            rt1 = (end - 1) >> 4
            brow = ber[sl] >> 4
            m1 = jnp.logical_and(cme > 0, (l0 & 15) != 0)
            m2 = jnp.logical_and(cme > 0, rt1 >= brow)
            dvec = db * 16 + lane
            plsc.store_scatter(
                rowidx, [dvec * 2],
                jnp.where(m1, rt0, jnp.full((16,), DUMPROW, jnp.int32)))
            plsc.store_scatter(
                rowidx, [dvec * 2 + 1],
                jnp.where(m2, rt1, jnp.full((16,), DUMPROW, jnp.int32)))

        def zrow(r, _):
            obk[r] = jnp.zeros((16,), jnp.int32)
            obv[r] = jnp.zeros((16,), jnp.float32)
            return 0

        lax.fori_loop(0, OBR, zrow, 0)
        pltpu.sync_copy(obk, kb_out.at[rowidx])
        pltpu.sync_copy(obv, pv_out.at[rowidx])

        dump16 = jnp.full((16,), DUMP, jnp.int32)

        def sdump(r, _):
            sp[r, pl.ds(0, 16)] = dump16
            sp[r, pl.ds(16, 16)] = dump16
            return 0

        lax.fori_loop(0, R, sdump, 0)

        def win_body(w, _):
            pltpu.sync_copy(kb_hbm.at[pl.ds(base + w * WIN, WIN)], wk)
            pltpu.sync_copy(pv_hbm.at[pl.ds(base + w * WIN, WIN)], wv)
            pltpu.sync_copy(
                whist_hbm.at[pl.ds((wid * NWIN + w) * R, R)], hcr)

            def tbl_body(db, carry):
                sl = pl.ds(db * 16, 16)
                p0 = loffr[sl]
                p1 = p0 + hcr[sl]
                arow = aer[sl] >> 4
                brow = ber[sl] >> 4
                u = jnp.maximum(p0 >> 4, arow)
                v = jnp.minimum(p1 >> 4, brow)
                m = jnp.maximum(v - u, 0)
                ob = carry + jnp.cumsum(m) - m
                vvr[sl] = v
                omur[sl] = ob - u
                ccomp = jnp.logical_and(
                    jnp.logical_and((p0 & 15) != 0, (p0 >> 4) >= arow),
                    (p0 >> 4) < v)
                dvec = db * 16 + lane
                for j in range(16):
                    jv = jnp.full((16,), j, jnp.int32)
                    ckv = plsc.load_gather(crk, [dvec, jv])
                    cvv = plsc.load_gather(crv, [dvec, jv])
                    plsc.store_scatter(obk, [ob, jv], ckv, mask=ccomp)
                    plsc.store_scatter(obv, [ob, jv], cvv, mask=ccomp)
                return carry + jnp.sum(m)

            lax.fori_loop(0, R // 16, tbl_body, 0)

            def vreg_body(i, _):
                x = wk[pl.ds(i * 16, 16)]
                val = wv[pl.ds(i * 16, 16)]
                d = lax.shift_right_logical(x, shift) & (R - 1)
                cnt, last = plsc.scan_count(d)
                off = plsc.load_gather(loffr, [d])
                pos = off + cnt - 1
                plsc.store_scatter(loffr, [d], off + cnt, mask=last)
                ridx = pos >> 4
                slot = pos & 15
                ae = plsc.load_gather(aer, [d])
                be = plsc.load_gather(ber, [d])
                me = jnp.logical_or(pos < ae, pos >= be)
                v = plsc.load_gather(vvr, [d])
                nme = jnp.logical_not(me)
                mout = jnp.logical_and(nme, ridx < v)
                mnc = jnp.logical_and(nme, ridx >= v)
                omu = plsc.load_gather(omur, [d])
                orow = omu + ridx
                plsc.store_scatter(obk, [orow, slot], x, mask=mout)
                plsc.store_scatter(obv, [orow, slot], val, mask=mout)
                plsc.store_scatter(rowidx, [orow], ridx, mask=mout)
                plsc.store_scatter(crk, [d, slot], x, mask=mnc)
                plsc.store_scatter(crv, [d, slot], val, mask=mnc)
                l0 = plsc.load_gather(loff0r, [d])
                sidx = jnp.where(pos < ae, pos - l0, 16 + pos - be)
                plsc.store_scatter(sp, [d, sidx], pos, mask=me)
                plsc.store_scatter(sk, [d, sidx], x, mask=me)
                plsc.store_scatter(sv, [d, sidx], val, mask=me)
                return 0

            lax.fori_loop(0, VPW, vreg_body, 0)
            pltpu.sync_copy(obk, kb_out.at[rowidx])
            pltpu.sync_copy(obv, pv_out.at[rowidx])
            return 0

        lax.fori_loop(0, NWIN, win_body, 0)
        pltpu.sync_copy(sp, sp_out.at[wid])
        pltpu.sync_copy(sk, sk_out.at[wid])
        pltpu.sync_copy(sv, sv_out.at[wid])

    return perm_k


_HIST0 = _hist0_kernel()
_PERM = [_permute_kernel(8 * p) for p in range(4)]
_CLEAN = [_cleanup_kernel(8 * p, True) for p in range(1, 4)] + [
    _cleanup_kernel(0, False)]


def kernel(keys, values):
    kb = _elementwise_tc(_pack_body, keys, jnp.int32)
    pv = values
    hist, whist = _HIST0(kb)
    for p in range(4):
        kb2, pv2, sp_, sk_, sv_ = _PERM[p](kb, pv, hist, whist)
        kb, pv, hist, whist = _CLEAN[p](
            kb2.reshape(-1), pv2.reshape(-1),
            sp_.reshape(-1), sk_.reshape(-1), sv_.reshape(-1))
    keys_out = _elementwise_tc(_unpack_body, kb[:N], jnp.float32)
    return keys_out, pv[:N]
